# Initial kernel scaffold; baseline (speedup 1.0000x reference)
import jax
import jax.numpy as jnp
from jax.experimental import pallas as pl


def kernel(y_pr, y_gt):
    raise NotImplementedError("write your pallas kernel here")


# speedup vs baseline: 29.9179x; 29.9179x over previous
import functools

import jax
import jax.numpy as jnp
from jax import lax
from jax.experimental import pallas as pl
from jax.experimental.pallas import tpu as pltpu
from jax.experimental.pallas import tpu_sc as plsc

EPS = 1e-07
_NC = 2
_NS = 16
_LANES = 16
_NW = _NC * _NS
_N = 32 * 512 * 512
_PER_W = _N // _NW
_CHUNK = 16384
_NCHUNK = _PER_W // _CHUNK
_VECS = _CHUNK // _LANES
_HCOLS = 512


def _hist_body(pr_hbm, gt_hbm, out_hbm, pr_buf, gt_buf, hist, sem0, sem1):
    wid = lax.axis_index("s") * _NC + lax.axis_index("c")
    base = wid * _PER_W
    sems = (sem0, sem1)

    def zbody(i, _):
        hist[pl.ds(i * _LANES, _LANES)] = jnp.zeros((_LANES,), jnp.float32)
        return 0
    lax.fori_loop(0, _LANES * _HCOLS // _LANES, zbody, 0)

    def copies(c, b):
        off = base + c * _CHUNK
        return (
            pltpu.make_async_copy(
                pr_hbm.at[pl.ds(off, _CHUNK)], pr_buf.at[b], sems[b]),
            pltpu.make_async_copy(
                gt_hbm.at[pl.ds(off, _CHUNK)], gt_buf.at[b], sems[b]),
        )

    lane_base = lax.broadcasted_iota(jnp.int32, (_LANES,), 0) * _HCOLS
    ones = jnp.ones((_LANES,), jnp.float32)

    for cp in copies(0, 0):
        cp.start()
    for c in range(_NCHUNK):
        b = c % 2
        if c + 1 < _NCHUNK:
            for cp in copies(c + 1, (c + 1) % 2):
                cp.start()
        for cp in copies(c, b):
            cp.wait()

        def body(i, _, b=b):
            x = pr_buf[b, pl.ds(i * _LANES, _LANES)]
            g = gt_buf[b, pl.ds(i * _LANES, _LANES)]
            pr255 = 255.0 / (1.0 + jnp.exp(-x))
            bin_i = jnp.clip(pr255.astype(jnp.int32), 0, 254)
            pos = (g * 255.0) > 128.0
            col = lane_base + (254 - bin_i) + jnp.where(pos, 0, 256)
            plsc.addupdate_scatter(hist, [col], ones)
            return 0

        lax.fori_loop(0, _VECS, body, 0, unroll=4)

    pltpu.sync_copy(hist, out_hbm.at[wid])


_hist_call = functools.partial(
    pl.kernel,
    out_type=jax.ShapeDtypeStruct((_NW, _LANES * _HCOLS), jnp.float32),
    mesh=plsc.VectorSubcoreMesh(core_axis_name="c", subcore_axis_name="s"),
    compiler_params=pltpu.CompilerParams(needs_layout_passes=False),
    scratch_types=[
        pltpu.VMEM((2, _CHUNK), jnp.float32),
        pltpu.VMEM((2, _CHUNK), jnp.float32),
        pltpu.VMEM((_LANES * _HCOLS,), jnp.float32),
        pltpu.SemaphoreType.DMA,
        pltpu.SemaphoreType.DMA,
    ],
)(_hist_body)


def _aiu_body(parts_ref, out_ref):
    hsum = jnp.sum(parts_ref[...], axis=0)
    pp = hsum[0:255]
    nn = hsum[256:511]
    gt_num = jnp.sum(pp)

    def cum(x):
        for k in (1, 2, 4, 8, 16, 32, 64, 128):
            x = x + jnp.concatenate([jnp.zeros((k,), jnp.float32), x[:-k]])
        return x

    ppc = cum(pp)
    nnc = cum(nn)
    denom = gt_num + nnc + EPS
    out_ref[...] = jnp.where(gt_num == 0.0, ppc + EPS / denom, ppc / denom)


def kernel(y_pr, y_gt):
    parts = _hist_call(y_pr.reshape(_N), y_gt.reshape(_N))
    return pl.pallas_call(
        _aiu_body,
        out_shape=jax.ShapeDtypeStruct((255,), jnp.float32),
    )(parts.reshape(_NW * _LANES, _HCOLS))


# speedup vs baseline: 101.0131x; 3.3763x over previous
import functools

import jax
import jax.numpy as jnp
from jax import lax
from jax.experimental import pallas as pl
from jax.experimental.pallas import tpu as pltpu
from jax.experimental.pallas import tpu_sc as plsc

EPS = 1e-07
_NC = 2
_NS = 16
_LANES = 16
_NW = _NC * _NS
_N = 32 * 512 * 512
_PER_W = _N // _NW
_CHUNK = 16384
_NCHUNK = _PER_W // _CHUNK
_VECS = _CHUNK // _LANES
_HCOLS = 512


def _hist_body(pr_hbm, gt_hbm, out_hbm, pr_buf, gt_buf, hist, sem0, sem1):
    wid = lax.axis_index("s") * _NC + lax.axis_index("c")
    base = wid * _PER_W
    sems = (sem0, sem1)

    def zbody(i, _):
        hist[pl.ds(i * _LANES, _LANES)] = jnp.zeros((_LANES,), jnp.float32)
        return 0
    lax.fori_loop(0, _LANES * _HCOLS // _LANES, zbody, 0)

    def copies(c, b):
        off = base + c * _CHUNK
        return (
            pltpu.make_async_copy(
                pr_hbm.at[pl.ds(off, _CHUNK)], pr_buf.at[b], sems[b]),
            pltpu.make_async_copy(
                gt_hbm.at[pl.ds(off, _CHUNK)], gt_buf.at[b], sems[b]),
        )

    lane_base = lax.broadcasted_iota(jnp.int32, (_LANES,), 0) * _HCOLS
    ones = jnp.ones((_LANES,), jnp.float32)

    for cp in copies(0, 0):
        cp.start()
    for c in range(_NCHUNK):
        b = c % 2
        if c + 1 < _NCHUNK:
            for cp in copies(c + 1, (c + 1) % 2):
                cp.start()
        for cp in copies(c, b):
            cp.wait()

        @plsc.parallel_loop(0, _VECS, unroll=8)
        def body(i, b=b):
            x = pr_buf[b, pl.ds(i * _LANES, _LANES)]
            g = gt_buf[b, pl.ds(i * _LANES, _LANES)]
            pr255 = 255.0 / (1.0 + jnp.exp(-x))
            bin_i = jnp.clip(pr255.astype(jnp.int32), 0, 254)
            pos = (g * 255.0) > 128.0
            col = lane_base + (254 - bin_i) + jnp.where(pos, 0, 256)
            plsc.addupdate_scatter(hist, [col], ones)

    pltpu.sync_copy(hist, out_hbm.at[wid])


_hist_call = functools.partial(
    pl.kernel,
    out_type=jax.ShapeDtypeStruct((_NW, _LANES * _HCOLS), jnp.float32),
    mesh=plsc.VectorSubcoreMesh(core_axis_name="c", subcore_axis_name="s"),
    compiler_params=pltpu.CompilerParams(needs_layout_passes=False),
    scratch_types=[
        pltpu.VMEM((2, _CHUNK), jnp.float32),
        pltpu.VMEM((2, _CHUNK), jnp.float32),
        pltpu.VMEM((_LANES * _HCOLS,), jnp.float32),
        pltpu.SemaphoreType.DMA,
        pltpu.SemaphoreType.DMA,
    ],
)(_hist_body)


def _aiu_body(parts_ref, out_ref):
    hsum = jnp.sum(parts_ref[...], axis=0)
    pp = hsum[0:255]
    nn = hsum[256:511]
    gt_num = jnp.sum(pp)

    def cum(x):
        for k in (1, 2, 4, 8, 16, 32, 64, 128):
            x = x + jnp.concatenate([jnp.zeros((k,), jnp.float32), x[:-k]])
        return x

    ppc = cum(pp)
    nnc = cum(nn)
    denom = gt_num + nnc + EPS
    out_ref[...] = jnp.where(gt_num == 0.0, ppc + EPS / denom, ppc / denom)


def kernel(y_pr, y_gt):
    parts = _hist_call(y_pr.reshape(_N), y_gt.reshape(_N))
    return pl.pallas_call(
        _aiu_body,
        out_shape=jax.ShapeDtypeStruct((255,), jnp.float32),
    )(parts.reshape(_NW * _LANES, _HCOLS))


# speedup vs baseline: 179.4435x; 1.7764x over previous
import functools

import jax
import jax.numpy as jnp
from jax import lax
from jax.experimental import pallas as pl
from jax.experimental.pallas import tpu as pltpu
from jax.experimental.pallas import tpu_sc as plsc

EPS = 1e-07
_NC = 2
_NS = 16
_LANES = 16
_NW = _NC * _NS
_N = 32 * 512 * 512
_PER_W = _N // _NW
_ROWS = 32
_CHUNK = _ROWS * 512
_NCHUNK = _PER_W // _CHUNK
_VECS = _CHUNK // _LANES
_HCOLS = 512


def _hist_body(pr_hbm, gt_hbm, out_hbm, pr_buf, gt_buf, hist, sem0, sem1):
    wid = lax.axis_index("s") * _NC + lax.axis_index("c")
    sems = (sem0, sem1)

    def zbody(i, _):
        hist[pl.ds(i * _LANES, _LANES)] = jnp.zeros((_LANES,), jnp.float32)
        return 0
    lax.fori_loop(0, _LANES * _HCOLS // _LANES, zbody, 0)

    def copies(c, b):
        rows = pl.ds(c * _ROWS, _ROWS)
        return (
            pltpu.make_async_copy(
                pr_hbm.at[wid, 0, rows, :], pr_buf.at[b], sems[b]),
            pltpu.make_async_copy(
                gt_hbm.at[wid, 0, rows, :], gt_buf.at[b], sems[b]),
        )

    lane_base = lax.broadcasted_iota(jnp.int32, (_LANES,), 0) * _HCOLS
    ones = jnp.ones((_LANES,), jnp.float32)

    for cp in copies(0, 0):
        cp.start()
    for c in range(_NCHUNK):
        b = c % 2
        if c + 1 < _NCHUNK:
            for cp in copies(c + 1, (c + 1) % 2):
                cp.start()
        for cp in copies(c, b):
            cp.wait()

        @plsc.parallel_loop(0, _VECS, unroll=8)
        def body(i, b=b):
            r = i // (512 // _LANES)
            cc = (i % (512 // _LANES)) * _LANES
            x = pr_buf[b, r, pl.ds(cc, _LANES)]
            g = gt_buf[b, r, pl.ds(cc, _LANES)]
            pr255 = 255.0 / (1.0 + jnp.exp(-x))
            bin_i = jnp.clip(pr255.astype(jnp.int32), 0, 254)
            pos = (g * 255.0) > 128.0
            hidx = lane_base + (254 - bin_i) + jnp.where(pos, 0, 256)
            plsc.addupdate_scatter(hist, [hidx], ones)

    pltpu.sync_copy(hist, out_hbm.at[wid])


_hist_call = functools.partial(
    pl.kernel,
    out_type=jax.ShapeDtypeStruct((_NW, _LANES * _HCOLS), jnp.float32),
    mesh=plsc.VectorSubcoreMesh(core_axis_name="c", subcore_axis_name="s"),
    compiler_params=pltpu.CompilerParams(needs_layout_passes=False),
    scratch_types=[
        pltpu.VMEM((2, _ROWS, 512), jnp.float32),
        pltpu.VMEM((2, _ROWS, 512), jnp.float32),
        pltpu.VMEM((_LANES * _HCOLS,), jnp.float32),
        pltpu.SemaphoreType.DMA,
        pltpu.SemaphoreType.DMA,
    ],
)(_hist_body)


def _aiu_body(parts_ref, out_ref):
    hsum = jnp.sum(parts_ref[...], axis=0)
    pp = hsum[0:255]
    nn = hsum[256:511]
    gt_num = jnp.sum(pp)

    def cum(x):
        for k in (1, 2, 4, 8, 16, 32, 64, 128):
            x = x + jnp.concatenate([jnp.zeros((k,), jnp.float32), x[:-k]])
        return x

    ppc = cum(pp)
    nnc = cum(nn)
    denom = gt_num + nnc + EPS
    out_ref[...] = jnp.where(gt_num == 0.0, ppc + EPS / denom, ppc / denom)


def kernel(y_pr, y_gt):
    parts = _hist_call(y_pr, y_gt)
    return pl.pallas_call(
        _aiu_body,
        out_shape=jax.ShapeDtypeStruct((255,), jnp.float32),
    )(parts.reshape(_NW * _LANES, _HCOLS))


# speedup vs baseline: 194.0279x; 1.0813x over previous
import functools

import jax
import jax.numpy as jnp
from jax import lax
from jax.experimental import pallas as pl
from jax.experimental.pallas import tpu as pltpu
from jax.experimental.pallas import tpu_sc as plsc

EPS = 1e-07
_NC = 2
_NS = 16
_LANES = 16
_NW = _NC * _NS
_N = 32 * 512 * 512
_PER_W = _N // _NW
_ROWS = 32
_CHUNK = _ROWS * 512
_NCHUNK = _PER_W // _CHUNK
_VECS = _CHUNK // _LANES
_HCOLS = 512


def _hist_body(pr_hbm, gt_hbm, out_hbm, pr_buf, gt_buf, hist, sem0, sem1):
    wid = lax.axis_index("s") * _NC + lax.axis_index("c")
    sems = (sem0, sem1)

    def zbody(i, _):
        hist[pl.ds(i * _LANES, _LANES)] = jnp.zeros((_LANES,), jnp.float32)
        return 0
    lax.fori_loop(0, _LANES * _HCOLS // _LANES, zbody, 0)

    def copies(c, b):
        rows = pl.ds(c * _ROWS, _ROWS)
        return (
            pltpu.make_async_copy(
                pr_hbm.at[wid, 0, rows, :], pr_buf.at[b], sems[b]),
            pltpu.make_async_copy(
                gt_hbm.at[wid, 0, rows, :], gt_buf.at[b], sems[b]),
        )

    lane_base = lax.broadcasted_iota(jnp.int32, (_LANES,), 0) * _HCOLS
    ones = jnp.ones((_LANES,), jnp.float32)

    for c0 in range(2):
        for cp in copies(c0, c0):
            cp.start()

    @pl.loop(0, _NCHUNK, step=2)
    def chunk_loop(c):
        for b in range(2):
            cc = c + b
            for cp in copies(cc, b):
                cp.wait()

            @plsc.parallel_loop(0, _VECS, unroll=16)
            def body(i, b=b):
                r = i // (512 // _LANES)
                cv = (i % (512 // _LANES)) * _LANES
                x = pr_buf[b, r, pl.ds(cv, _LANES)]
                g = gt_buf[b, r, pl.ds(cv, _LANES)]
                pr255 = 255.0 / (1.0 + jnp.exp(-x))
                bin_i = jnp.clip(pr255.astype(jnp.int32), 0, 254)
                pos = (g * 255.0) > 128.0
                hidx = lane_base + (254 - bin_i) + jnp.where(pos, 0, 256)
                plsc.addupdate_scatter(hist, [hidx], ones)

            @pl.when(cc + 2 < _NCHUNK)
            def _prefetch(cc=cc, b=b):
                for cp in copies(cc + 2, b):
                    cp.start()

    pltpu.sync_copy(hist, out_hbm.at[wid])


_hist_call = functools.partial(
    pl.kernel,
    out_type=jax.ShapeDtypeStruct((_NW, _LANES * _HCOLS), jnp.float32),
    mesh=plsc.VectorSubcoreMesh(core_axis_name="c", subcore_axis_name="s"),
    compiler_params=pltpu.CompilerParams(needs_layout_passes=False),
    scratch_types=[
        pltpu.VMEM((2, _ROWS, 512), jnp.float32),
        pltpu.VMEM((2, _ROWS, 512), jnp.float32),
        pltpu.VMEM((_LANES * _HCOLS,), jnp.float32),
        pltpu.SemaphoreType.DMA,
        pltpu.SemaphoreType.DMA,
    ],
)(_hist_body)


def _aiu_body(parts_ref, out_ref):
    hsum = jnp.sum(parts_ref[...], axis=0)
    pp = hsum[0:255]
    nn = hsum[256:511]
    gt_num = jnp.sum(pp)

    def cum(x):
        for k in (1, 2, 4, 8, 16, 32, 64, 128):
            x = x + jnp.concatenate([jnp.zeros((k,), jnp.float32), x[:-k]])
        return x

    ppc = cum(pp)
    nnc = cum(nn)
    denom = gt_num + nnc + EPS
    out_ref[...] = jnp.where(gt_num == 0.0, ppc + EPS / denom, ppc / denom)


def kernel(y_pr, y_gt):
    parts = _hist_call(y_pr, y_gt)
    return pl.pallas_call(
        _aiu_body,
        out_shape=jax.ShapeDtypeStruct((255,), jnp.float32),
    )(parts.reshape(_NW * _LANES, _HCOLS))
